# baseline (speedup 1.0000x reference)
import functools

import jax
import jax.numpy as jnp
from jax import lax
from jax.experimental import pallas as pl
from jax.experimental.pallas import tpu as pltpu
from jax.experimental.pallas import tpu_sc as plsc

_NE = 1024
_ED = 64
_B = 8
_HW = 1024
_N = _B * _HW
_CC = 0.25


def _dist_kernel(x_ref, w_ref, idx_ref, loss_ref, perp_ref, cnt_ref, acc_ref):
    w = w_ref[...]
    wsq = jnp.sum((w * w).T, axis=0)
    cnt_ref[...] = jnp.zeros_like(cnt_ref)
    acc_ref[0, 0] = 0.0

    def body(i, _):
        x = x_ref[i]
        flat = x.T
        flatsq = jnp.sum(x * x, axis=0)[:, None]
        m = lax.dot_general(flat, w, (((1,), (1,)), ((), ())),
                            preferred_element_type=jnp.float32)
        d = (flatsq + wsq[None, :]) - 2.0 * m
        dmin = jnp.min(d, axis=1, keepdims=True)
        lane = lax.broadcasted_iota(jnp.int32, (_HW, _NE), 1)
        idx = jnp.min(jnp.where(d == dmin, lane, _NE), axis=1)
        idx_ref[0, i, :] = idx
        acc_ref[0, 0] += jnp.sum(dmin)
        cnt_ref[0, :] += jnp.sum(
            (lane == idx[:, None]).astype(jnp.float32), axis=0)
        return 0

    lax.fori_loop(0, _B, body, 0)
    mse = acc_ref[0, 0] / float(_N * _ED)
    loss_ref[0, 0] = mse + _CC * mse
    p = cnt_ref[0] * (1.0 / _N)
    perp_ref[0, 0] = jnp.exp(-jnp.sum(p * jnp.log(p + 1e-10)))


def _sc_out(wt, idx):
    nw = 32
    bpw = _N // nw
    L = 16

    @functools.partial(
        pl.kernel,
        mesh=plsc.VectorSubcoreMesh(core_axis_name="c", subcore_axis_name="s"),
        compiler_params=pltpu.CompilerParams(needs_layout_passes=False),
        out_type=jax.ShapeDtypeStruct((_B, _ED, _HW), jnp.float32),
        scratch_types=[
            pltpu.VMEM((bpw,), jnp.int32),
            pltpu.VMEM((_ED * _NE,), jnp.float32),
            pltpu.VMEM((_ED, bpw), jnp.float32),
        ],
    )
    def out_k(wt_hbm, idx_hbm, out_hbm, idx_v, wt_v, out_v):
        wid = lax.axis_index("s") * 2 + lax.axis_index("c")
        b = wid // 4
        c0 = (wid % 4) * bpw
        pltpu.sync_copy(wt_hbm, wt_v)
        pltpu.sync_copy(idx_hbm.at[pl.ds(b * _HW + c0, bpw)], idx_v)

        def body(k, _):
            iv = idx_v[pl.ds(k * L, L)]
            for c in range(_ED):
                qv = plsc.load_gather(wt_v, [iv + (c * _NE)])
                out_v[c, pl.ds(k * L, L)] = qv
            return 0

        lax.fori_loop(0, bpw // L, body, 0)
        pltpu.sync_copy(out_v, out_hbm.at[b, :, pl.ds(c0, bpw)])

    return out_k(wt, idx)


def kernel(inputs, W):
    x3 = inputs.reshape(_B, _ED, _HW)
    idx3, loss, perp = pl.pallas_call(
        _dist_kernel,
        in_specs=[pl.BlockSpec(memory_space=pltpu.VMEM),
                  pl.BlockSpec(memory_space=pltpu.VMEM)],
        out_specs=[pl.BlockSpec(memory_space=pltpu.VMEM),
                   pl.BlockSpec(memory_space=pltpu.SMEM),
                   pl.BlockSpec(memory_space=pltpu.SMEM)],
        out_shape=[jax.ShapeDtypeStruct((1, _B, _HW), jnp.int32),
                   jax.ShapeDtypeStruct((1, 1), jnp.float32),
                   jax.ShapeDtypeStruct((1, 1), jnp.float32)],
        scratch_shapes=[pltpu.VMEM((1, _NE), jnp.float32),
                        pltpu.SMEM((1, 1), jnp.float32)],
    )(x3, W)
    out3 = _sc_out(W.T.reshape(_ED * _NE), idx3.reshape(_N))
    return out3.reshape(8, 64, 32, 32), loss[0, 0], perp[0, 0]
